# baseline (speedup 1.0000x reference)
import functools

import jax
import jax.numpy as jnp
from jax import lax
from jax.experimental import pallas as pl
from jax.experimental.pallas import tpu as pltpu
from jax.experimental.pallas import tpu_sc as plsc

N_IN = 5000
N_PAD = 5120
N_OUT = 10000
E = 320000
F = 128
CHUNK = 128
N_CHUNKS = E // CHUNK
MAX_CHUNKS_PER_W = -(-N_CHUNKS // 32)
STRIPE = 624
STRIPE_LAST = N_OUT - 15 * STRIPE
UROWS = N_PAD // 32
UCHUNK = 80


def _sc_mesh():
    return plsc.VectorSubcoreMesh(
        core_axis_name="c", subcore_axis_name="s", num_cores=2, num_subcores=16
    )


@functools.partial(
    pl.kernel,
    out_type=jax.ShapeDtypeStruct((2 * N_OUT, F), jnp.float32),
    mesh=_sc_mesh(),
    scratch_types=[
        pltpu.VMEM((CHUNK,), jnp.int32),
        pltpu.VMEM((CHUNK,), jnp.int32),
        pltpu.VMEM((CHUNK,), jnp.int32),
        pltpu.VMEM((CHUNK,), jnp.int32),
        pltpu.VMEM((CHUNK,), jnp.int32),
        pltpu.VMEM((CHUNK,), jnp.int32),
        pltpu.VMEM((CHUNK, F), jnp.float32),
        pltpu.VMEM((CHUNK, F), jnp.float32),
        pltpu.VMEM((CHUNK, F), jnp.float32),
        pltpu.VMEM_SHARED((N_OUT, F), jnp.float32),
        pltpu.SemaphoreType.DMA,
        pltpu.SemaphoreType.DMA,
        pltpu.SemaphoreType.DMA,
        pltpu.SemaphoreType.DMA,
        pltpu.SemaphoreType.DMA,
        pltpu.SemaphoreType.DMA,
        pltpu.SemaphoreType.DMA,
        pltpu.SemaphoreType.DMA,
        pltpu.SemaphoreType.DMA,
    ],
)
def _spmm_sc(table_hbm, src_hbm, dst_hbm, zero_hbm, out_hbm,
             src_v0, src_v1, src_v2, dst_v0, dst_v1, dst_v2,
             rows0_v, rows1_v, rows2_v, acc_sh,
             isem0, isem1, isem2, gsem0, gsem1, gsem2, ssem0, ssem1, ssem2):
    c = lax.axis_index("c")
    s = lax.axis_index("s")
    w = c * 16 + s
    n_w = 78 + jnp.where(w < 4, 1, 0)
    srcs = (src_v0, src_v1, src_v2)
    dsts = (dst_v0, dst_v1, dst_v2)
    rows = (rows0_v, rows1_v, rows2_v)
    isems = (isem0, isem1, isem2)
    gsems = (gsem0, gsem1, gsem2)
    ssems = (ssem0, ssem1, ssem2)

    def fire_idx(j, b):
        base = (w + 32 * j) * CHUNK
        pltpu.async_copy(src_hbm.at[pl.ds(base, CHUNK)], srcs[b], isems[b])
        pltpu.async_copy(dst_hbm.at[pl.ds(base, CHUNK)], dsts[b], isems[b])

    def wait_idx(j, b):
        base = (w + 32 * j) * CHUNK
        pltpu.make_async_copy(src_hbm.at[pl.ds(base, CHUNK)], srcs[b], isems[b]).wait()
        pltpu.make_async_copy(dst_hbm.at[pl.ds(base, CHUNK)], dsts[b], isems[b]).wait()

    def fire_gather(b):
        pltpu.async_copy(table_hbm.at[srcs[b]], rows[b], gsems[b])

    def wait_gather(b):
        pltpu.make_async_copy(table_hbm.at[srcs[b]], rows[b], gsems[b]).wait()

    def fire_scatter(b):
        pltpu.async_copy(rows[b], acc_sh.at[dsts[b]], ssems[b], add=True)

    def wait_scatter(b):
        pltpu.make_async_copy(rows[b], acc_sh.at[dsts[b]], ssems[b]).wait()


    @pl.when(s < 15)
    def _():
        pltpu.sync_copy(zero_hbm.at[pl.ds(0, STRIPE)],
                        acc_sh.at[pl.ds(s * STRIPE, STRIPE)])

    @pl.when(s == 15)
    def _():
        pltpu.sync_copy(zero_hbm, acc_sh.at[pl.ds(15 * STRIPE, STRIPE_LAST)])

    plsc.subcore_barrier()

    fire_idx(0, 0)
    fire_idx(1, 1)
    wait_idx(0, 0)
    fire_gather(0)

    def step(j, b):
        wait_gather(b)

        @pl.when(j >= 1)
        def _():
            wait_scatter((b + 2) % 3)

        @pl.when(j + 2 < n_w)
        def _():
            fire_idx(j + 2, (b + 2) % 3)

        fire_scatter(b)

        @pl.when(j + 1 < n_w)
        def _():
            wait_idx(j + 1, (b + 1) % 3)
            fire_gather((b + 1) % 3)

    def body(i, carry):
        j0 = 3 * i
        for t in range(3):
            @pl.when(j0 + t < n_w)
            def _():
                step(j0 + t, t)

        return carry

    lax.fori_loop(0, 27, body, 0)

    @pl.when(w < 4)
    def _():
        wait_scatter(78 % 3)

    @pl.when(w >= 4)
    def _():
        wait_scatter(77 % 3)

    plsc.subcore_barrier()

    @pl.when(s < 15)
    def _():
        pltpu.sync_copy(
            acc_sh.at[pl.ds(s * STRIPE, STRIPE)],
            out_hbm.at[pl.ds(c * N_OUT + s * STRIPE, STRIPE)],
        )

    @pl.when(s == 15)
    def _():
        pltpu.sync_copy(
            acc_sh.at[pl.ds(15 * STRIPE, STRIPE_LAST)],
            out_hbm.at[pl.ds(c * N_OUT + 15 * STRIPE, STRIPE_LAST)],
        )


@functools.partial(
    pl.kernel,
    out_type=[
        jax.ShapeDtypeStruct((N_PAD, F), jnp.float32),
        jax.ShapeDtypeStruct((N_PAD, F), jnp.float32),
    ],
    mesh=_sc_mesh(),
    scratch_types=[
        pltpu.VMEM((UCHUNK,), jnp.int32),
        pltpu.VMEM((UCHUNK,), jnp.int32),
        pltpu.VMEM((UCHUNK, F), jnp.float32),
        pltpu.VMEM((UCHUNK, F), jnp.float32),
        pltpu.SemaphoreType.DMA,
        pltpu.SemaphoreType.DMA,
    ],
)
def _unpool_sc(y_hbm, yl_hbm, u0_hbm, u1_hbm, ny_hbm, nyl_hbm,
               idx0_v, idx1_v, b0_v, b1_v, sem0, sem1):
    c = lax.axis_index("c")
    s = lax.axis_index("s")
    w = c * 16 + s

    def do_chunk(t_hbm, o_hbm, base):
        pltpu.sync_copy(u0_hbm.at[pl.ds(base, UCHUNK)], idx0_v)
        pltpu.sync_copy(u1_hbm.at[pl.ds(base, UCHUNK)], idx1_v)
        cp0 = pltpu.async_copy(t_hbm.at[idx0_v], b0_v, sem0)
        cp1 = pltpu.async_copy(t_hbm.at[idx1_v], b1_v, sem1)
        cp0.wait()
        cp1.wait()

        def row(i, carry):
            for jj in range(F // 16):
                sl = pl.ds(jj * 16, 16)
                b0_v[i, sl] = (b0_v[i, sl] + b1_v[i, sl]) * 0.5
            return carry

        lax.fori_loop(0, UCHUNK, row, 0)
        pltpu.sync_copy(b0_v, o_hbm.at[pl.ds(base, UCHUNK)])

    for k in range(UROWS // UCHUNK):
        base = w * UROWS + k * UCHUNK
        do_chunk(y_hbm, ny_hbm, base)
        do_chunk(yl_hbm, nyl_hbm, base)


ATT_R = 1024


def _attn_body(mvb_ref, mvf_ref, wq_ref, wk_ref, wv_ref, o_ref):
    q = jnp.dot(mvb_ref[...], wq_ref[...], preferred_element_type=jnp.float32, precision=lax.Precision.HIGHEST)
    k = jnp.dot(mvf_ref[...], wk_ref[...], preferred_element_type=jnp.float32, precision=lax.Precision.HIGHEST)
    v = jnp.dot(mvf_ref[...], wv_ref[...], preferred_element_type=jnp.float32, precision=lax.Precision.HIGHEST)
    s = lax.dot_general(q.astype(jnp.bfloat16), k.astype(jnp.bfloat16),
                        (((1,), (1,)), ((), ())),
                        preferred_element_type=jnp.float32) * (1.0 / jnp.sqrt(8.0))
    col = lax.broadcasted_iota(jnp.int32, s.shape, 1)
    s = jnp.where(col < N_IN, s, -1e30)
    m = jnp.max(s, axis=1, keepdims=True)
    p = jnp.exp(s - m)
    denom = jnp.sum(p, axis=1, keepdims=True)
    att = jnp.dot(p.astype(jnp.bfloat16), v.astype(jnp.bfloat16),
                  preferred_element_type=jnp.float32)
    o_ref[...] = att / denom


def _attention(mv):
    return pl.pallas_call(
        _attn_body,
        grid=(N_PAD // ATT_R,),
        in_specs=[
            pl.BlockSpec((ATT_R, 8), lambda i: (i, 0)),
            pl.BlockSpec((N_PAD, 8), lambda i: (0, 0)),
            pl.BlockSpec((8, 8), lambda i: (0, 0)),
            pl.BlockSpec((8, 8), lambda i: (0, 0)),
            pl.BlockSpec((8, 8), lambda i: (0, 0)),
        ],
        out_specs=pl.BlockSpec((ATT_R, 8), lambda i: (i, 0)),
        out_shape=jax.ShapeDtypeStruct((N_PAD, 8), jnp.float32),
    )


PRE_R = 1024


def _pre_body(cat_ref, w_ref, wl_ref, y_ref, yl_ref):
    cat = cat_ref[...]
    y_ref[...] = jnp.dot(cat, w_ref[...], preferred_element_type=jnp.float32, precision=lax.Precision.HIGHEST)
    yl_ref[...] = jnp.dot(cat, wl_ref[...], preferred_element_type=jnp.float32, precision=lax.Precision.HIGHEST)


def _pre(catp, c1_W, c1_Wl):
    in_dim = catp.shape[1]
    return pl.pallas_call(
        _pre_body,
        grid=(N_PAD // PRE_R,),
        in_specs=[
            pl.BlockSpec((PRE_R, in_dim), lambda i: (i, 0)),
            pl.BlockSpec((in_dim, F), lambda i: (0, 0)),
            pl.BlockSpec((in_dim, F), lambda i: (0, 0)),
        ],
        out_specs=[
            pl.BlockSpec((PRE_R, F), lambda i: (i, 0)),
            pl.BlockSpec((PRE_R, F), lambda i: (i, 0)),
        ],
        out_shape=[
            jax.ShapeDtypeStruct((N_PAD, F), jnp.float32),
            jax.ShapeDtypeStruct((N_PAD, F), jnp.float32),
        ],
    )(catp, c1_W, c1_Wl)


GC_R = 1000
GC_GRID = N_OUT // GC_R


def _p_specs():
    return [
        pl.BlockSpec((GC_R, F), lambda i: (i, 0)),
        pl.BlockSpec((GC_R, F), lambda i: (i + GC_GRID, 0)),
    ]


def _combine1_body(p0_ref, p1_ref, sl_ref, b_ref, o_ref):
    o_ref[...] = jnp.maximum(p0_ref[...] + p1_ref[...] + sl_ref[...] + b_ref[...], 0.0)


def _combine1(parts, s1l, b):
    return pl.pallas_call(
        _combine1_body,
        grid=(GC_GRID,),
        in_specs=_p_specs() + [
            pl.BlockSpec((GC_R, F), lambda i: (i, 0)),
            pl.BlockSpec((1, F), lambda i: (0, 0)),
        ],
        out_specs=pl.BlockSpec((GC_R, F), lambda i: (i, 0)),
        out_shape=jax.ShapeDtypeStruct((N_OUT, F), jnp.float32),
    )(parts, parts, s1l, b)


def _gconv_body(p0_ref, p1_ref, x_ref, w_ref, wl_ref, b_ref, o_ref, *, act):
    agg = p0_ref[...] + p1_ref[...]
    t = (jnp.dot(agg, w_ref[...], preferred_element_type=jnp.float32, precision=lax.Precision.HIGHEST)
         + jnp.dot(x_ref[...], wl_ref[...], preferred_element_type=jnp.float32, precision=lax.Precision.HIGHEST)
         + b_ref[...])
    if act:
        t = jnp.maximum(t, 0.0)
    o_ref[...] = t


def _gconv_res_body(p0_ref, p1_ref, x_ref, hp_ref, w_ref, wl_ref, b_ref, o_ref):
    agg = p0_ref[...] + p1_ref[...]
    t = (jnp.dot(agg, w_ref[...], preferred_element_type=jnp.float32, precision=lax.Precision.HIGHEST)
         + jnp.dot(x_ref[...], wl_ref[...], preferred_element_type=jnp.float32, precision=lax.Precision.HIGHEST)
         + b_ref[...])
    t = jnp.maximum(t, 0.0)
    o_ref[...] = (hp_ref[...] + t) * 0.5


def _gconv(parts, x, W, Wl, b, act=True, hprev=None):
    fo = W.shape[1]
    wspec = [
        pl.BlockSpec((F, fo), lambda i: (0, 0)),
        pl.BlockSpec((F, fo), lambda i: (0, 0)),
        pl.BlockSpec((1, fo), lambda i: (0, 0)),
    ]
    ospec = pl.BlockSpec((GC_R, fo), lambda i: (i, 0))
    oshape = jax.ShapeDtypeStruct((N_OUT, fo), jnp.float32)
    if hprev is None:
        return pl.pallas_call(
            functools.partial(_gconv_body, act=act),
            grid=(GC_GRID,),
            in_specs=_p_specs() + [pl.BlockSpec((GC_R, F), lambda i: (i, 0))] + wspec,
            out_specs=ospec,
            out_shape=oshape,
        )(parts, parts, x, W, Wl, b)
    return pl.pallas_call(
        _gconv_res_body,
        grid=(GC_GRID,),
        in_specs=_p_specs() + [
            pl.BlockSpec((GC_R, F), lambda i: (i, 0)),
            pl.BlockSpec((GC_R, F), lambda i: (i, 0)),
        ] + wspec,
        out_specs=ospec,
        out_shape=oshape,
    )(parts, parts, x, hprev, W, Wl, b)


def kernel(x, x2, x_hidden, edge_index, unpool_idx, Wq, Wk, Wv, c1_W, c1_Wl, c1_b,
           res_W, res_Wl, res_b, c2_W, c2_Wl, c2_b, out_W, out_Wl, out_b):
    src = edge_index[0]
    dst = edge_index[1]
    zeros640 = jnp.zeros((STRIPE_LAST, F), jnp.float32)

    mv = jnp.zeros((N_PAD, 8), jnp.float32).at[:N_IN, 1:4].set(x2[0])
    att = _attention(mv)(mv, mv, Wq, Wk, Wv)[:N_IN]

    cat = jnp.concatenate([x[0], x_hidden[0], att], axis=1)
    catp = jnp.pad(cat, ((0, N_PAD - N_IN), (0, 0)))
    y, yl = _pre(catp, c1_W, c1_Wl)

    u0 = jnp.pad(unpool_idx[:, 0], (0, N_PAD - N_IN))
    u1 = jnp.pad(unpool_idx[:, 1], (0, N_PAD - N_IN))
    ny, nyl = _unpool_sc(y, yl, u0, u1)
    s1 = jnp.concatenate([y[:N_IN], ny[:N_IN]], axis=0)
    s1l = jnp.concatenate([yl[:N_IN], nyl[:N_IN]], axis=0)

    def spmm(t):
        return _spmm_sc(t, src, dst, zeros640)

    h = _combine1(spmm(s1), s1l, c1_b[None, :])
    for i in range(6):
        t = _gconv(spmm(h), h, res_W[i, 0], res_Wl[i, 0], res_b[i, 0][None, :])
        h = _gconv(spmm(t), t, res_W[i, 1], res_Wl[i, 1], res_b[i, 1][None, :],
                   hprev=h)
    x4 = _gconv(spmm(h), h, c2_W, c2_Wl, c2_b[None, :])

    out_Wp = jnp.pad(out_W, ((0, 0), (0, F - out_W.shape[1])))
    out_Wlp = jnp.pad(out_Wl, ((0, 0), (0, F - out_Wl.shape[1])))
    out_bp = jnp.pad(out_b, (0, F - out_b.shape[0]))
    o = _gconv(spmm(x4), x4, out_Wp, out_Wlp, out_bp[None, :], act=False)
    return o[None, :, :3]


# speedup vs baseline: 1.1252x; 1.1252x over previous
import functools

import jax
import jax.numpy as jnp
from jax import lax
from jax.experimental import pallas as pl
from jax.experimental.pallas import tpu as pltpu
from jax.experimental.pallas import tpu_sc as plsc

N_IN = 5000
N_PAD = 5120
N_OUT = 10000
E = 320000
F = 128
CHUNK = 128
N_CHUNKS = E // CHUNK
MAX_CHUNKS_PER_W = -(-N_CHUNKS // 32)
STRIPE = 624
STRIPE_LAST = N_OUT - 15 * STRIPE
UROWS = N_PAD // 32
UCHUNK = 80


def _sc_mesh():
    return plsc.VectorSubcoreMesh(
        core_axis_name="c", subcore_axis_name="s", num_cores=2, num_subcores=16
    )


@functools.partial(
    pl.kernel,
    out_type=jax.ShapeDtypeStruct((2 * N_OUT, F), jnp.float32),
    mesh=_sc_mesh(),
    scratch_types=[
        pltpu.VMEM((CHUNK,), jnp.int32),
        pltpu.VMEM((CHUNK,), jnp.int32),
        pltpu.VMEM((CHUNK,), jnp.int32),
        pltpu.VMEM((CHUNK,), jnp.int32),
        pltpu.VMEM((CHUNK,), jnp.int32),
        pltpu.VMEM((CHUNK,), jnp.int32),
        pltpu.VMEM((CHUNK, F), jnp.float32),
        pltpu.VMEM((CHUNK, F), jnp.float32),
        pltpu.VMEM((CHUNK, F), jnp.float32),
        pltpu.VMEM_SHARED((N_OUT, F), jnp.float32),
        pltpu.SemaphoreType.DMA,
        pltpu.SemaphoreType.DMA,
        pltpu.SemaphoreType.DMA,
        pltpu.SemaphoreType.DMA,
        pltpu.SemaphoreType.DMA,
        pltpu.SemaphoreType.DMA,
    ],
)
def _spmm_sc(table_hbm, src_hbm, dst_hbm, zero_hbm, out_hbm,
             src_v0, src_v1, src_v2, dst_v0, dst_v1, dst_v2,
             rows0_v, rows1_v, rows2_v, acc_sh,
             isem0, isem1, isem2, gsem0, gsem1, gsem2):
    c = lax.axis_index("c")
    s = lax.axis_index("s")
    w = c * 16 + s
    n_w = 78 + jnp.where(w < 4, 1, 0)
    srcs = (src_v0, src_v1, src_v2)
    dsts = (dst_v0, dst_v1, dst_v2)
    rows = (rows0_v, rows1_v, rows2_v)
    isems = (isem0, isem1, isem2)
    gsems = (gsem0, gsem1, gsem2)

    def fire_idx(j, b):
        base = (w + 32 * j) * CHUNK
        pltpu.async_copy(src_hbm.at[pl.ds(base, CHUNK)], srcs[b], isems[b])
        pltpu.async_copy(dst_hbm.at[pl.ds(base, CHUNK)], dsts[b], isems[b])

    def wait_idx(j, b):
        base = (w + 32 * j) * CHUNK
        pltpu.make_async_copy(src_hbm.at[pl.ds(base, CHUNK)], srcs[b], isems[b]).wait()
        pltpu.make_async_copy(dst_hbm.at[pl.ds(base, CHUNK)], dsts[b], isems[b]).wait()

    def fire_gather(b):
        pltpu.async_copy(table_hbm.at[srcs[b]], rows[b], gsems[b])

    def wait_gather(b):
        pltpu.make_async_copy(table_hbm.at[srcs[b]], rows[b], gsems[b]).wait()

    def scatter(b):
        pltpu.sync_copy(rows[b], acc_sh.at[dsts[b]], add=True)


    @pl.when(s < 15)
    def _():
        pltpu.sync_copy(zero_hbm.at[pl.ds(0, STRIPE)],
                        acc_sh.at[pl.ds(s * STRIPE, STRIPE)])

    @pl.when(s == 15)
    def _():
        pltpu.sync_copy(zero_hbm, acc_sh.at[pl.ds(15 * STRIPE, STRIPE_LAST)])

    plsc.subcore_barrier()

    fire_idx(0, 0)
    fire_idx(1, 1)
    fire_idx(2, 2)
    wait_idx(0, 0)
    fire_gather(0)
    wait_idx(1, 1)
    fire_gather(1)

    def step(j, b):
        wait_gather(b)

        @pl.when(j + 2 < n_w)
        def _():
            wait_idx(j + 2, (b + 2) % 3)
            fire_gather((b + 2) % 3)

        scatter(b)

        @pl.when(j + 3 < n_w)
        def _():
            fire_idx(j + 3, b)

    def body(i, carry):
        j0 = 3 * i
        for t in range(3):
            @pl.when(j0 + t < n_w)
            def _():
                step(j0 + t, t)

        return carry

    lax.fori_loop(0, 27, body, 0)
    plsc.subcore_barrier()

    @pl.when(s < 15)
    def _():
        pltpu.sync_copy(
            acc_sh.at[pl.ds(s * STRIPE, STRIPE)],
            out_hbm.at[pl.ds(c * N_OUT + s * STRIPE, STRIPE)],
        )

    @pl.when(s == 15)
    def _():
        pltpu.sync_copy(
            acc_sh.at[pl.ds(15 * STRIPE, STRIPE_LAST)],
            out_hbm.at[pl.ds(c * N_OUT + 15 * STRIPE, STRIPE_LAST)],
        )


@functools.partial(
    pl.kernel,
    out_type=[
        jax.ShapeDtypeStruct((N_PAD, F), jnp.float32),
        jax.ShapeDtypeStruct((N_PAD, F), jnp.float32),
    ],
    mesh=_sc_mesh(),
    scratch_types=[
        pltpu.VMEM((UCHUNK,), jnp.int32),
        pltpu.VMEM((UCHUNK,), jnp.int32),
        pltpu.VMEM((UCHUNK, F), jnp.float32),
        pltpu.VMEM((UCHUNK, F), jnp.float32),
        pltpu.SemaphoreType.DMA,
        pltpu.SemaphoreType.DMA,
    ],
)
def _unpool_sc(y_hbm, yl_hbm, u0_hbm, u1_hbm, ny_hbm, nyl_hbm,
               idx0_v, idx1_v, b0_v, b1_v, sem0, sem1):
    c = lax.axis_index("c")
    s = lax.axis_index("s")
    w = c * 16 + s

    def do_chunk(t_hbm, o_hbm, base):
        pltpu.sync_copy(u0_hbm.at[pl.ds(base, UCHUNK)], idx0_v)
        pltpu.sync_copy(u1_hbm.at[pl.ds(base, UCHUNK)], idx1_v)
        cp0 = pltpu.async_copy(t_hbm.at[idx0_v], b0_v, sem0)
        cp1 = pltpu.async_copy(t_hbm.at[idx1_v], b1_v, sem1)
        cp0.wait()
        cp1.wait()

        def row(i, carry):
            for jj in range(F // 16):
                sl = pl.ds(jj * 16, 16)
                b0_v[i, sl] = (b0_v[i, sl] + b1_v[i, sl]) * 0.5
            return carry

        lax.fori_loop(0, UCHUNK, row, 0)
        pltpu.sync_copy(b0_v, o_hbm.at[pl.ds(base, UCHUNK)])

    for k in range(UROWS // UCHUNK):
        base = w * UROWS + k * UCHUNK
        do_chunk(y_hbm, ny_hbm, base)
        do_chunk(yl_hbm, nyl_hbm, base)


def _mm3(a, b):
    ah = a.astype(jnp.bfloat16)
    al = (a - ah.astype(jnp.float32)).astype(jnp.bfloat16)
    bh = b.astype(jnp.bfloat16)
    bl = (b - bh.astype(jnp.float32)).astype(jnp.bfloat16)

    def d(u, v):
        return jnp.dot(u, v, preferred_element_type=jnp.float32)

    return d(ah, bh) + (d(ah, bl) + d(al, bh))


ATT_R = 1024


def _attn_body(mvb_ref, mvf_ref, wq_ref, wk_ref, wv_ref, o_ref):
    q = _mm3(mvb_ref[...], wq_ref[...])
    k = _mm3(mvf_ref[...], wk_ref[...])
    v = _mm3(mvf_ref[...], wv_ref[...])
    s = lax.dot_general(q.astype(jnp.bfloat16), k.astype(jnp.bfloat16),
                        (((1,), (1,)), ((), ())),
                        preferred_element_type=jnp.float32) * (1.0 / jnp.sqrt(8.0))
    col = lax.broadcasted_iota(jnp.int32, s.shape, 1)
    s = jnp.where(col < N_IN, s, -1e30)
    m = jnp.max(s, axis=1, keepdims=True)
    p = jnp.exp(s - m)
    denom = jnp.sum(p, axis=1, keepdims=True)
    att = jnp.dot(p.astype(jnp.bfloat16), v.astype(jnp.bfloat16),
                  preferred_element_type=jnp.float32)
    o_ref[...] = att / denom


def _attention(mv):
    return pl.pallas_call(
        _attn_body,
        grid=(N_PAD // ATT_R,),
        in_specs=[
            pl.BlockSpec((ATT_R, 8), lambda i: (i, 0)),
            pl.BlockSpec((N_PAD, 8), lambda i: (0, 0)),
            pl.BlockSpec((8, 8), lambda i: (0, 0)),
            pl.BlockSpec((8, 8), lambda i: (0, 0)),
            pl.BlockSpec((8, 8), lambda i: (0, 0)),
        ],
        out_specs=pl.BlockSpec((ATT_R, 8), lambda i: (i, 0)),
        out_shape=jax.ShapeDtypeStruct((N_PAD, 8), jnp.float32),
    )


PRE_R = 1024


def _pre_body(cat_ref, w_ref, wl_ref, y_ref, yl_ref):
    cat = cat_ref[...]
    y_ref[...] = _mm3(cat, w_ref[...])
    yl_ref[...] = _mm3(cat, wl_ref[...])


def _pre(catp, c1_W, c1_Wl):
    in_dim = catp.shape[1]
    return pl.pallas_call(
        _pre_body,
        grid=(N_PAD // PRE_R,),
        in_specs=[
            pl.BlockSpec((PRE_R, in_dim), lambda i: (i, 0)),
            pl.BlockSpec((in_dim, F), lambda i: (0, 0)),
            pl.BlockSpec((in_dim, F), lambda i: (0, 0)),
        ],
        out_specs=[
            pl.BlockSpec((PRE_R, F), lambda i: (i, 0)),
            pl.BlockSpec((PRE_R, F), lambda i: (i, 0)),
        ],
        out_shape=[
            jax.ShapeDtypeStruct((N_PAD, F), jnp.float32),
            jax.ShapeDtypeStruct((N_PAD, F), jnp.float32),
        ],
    )(catp, c1_W, c1_Wl)


GC_R = 1000
GC_GRID = N_OUT // GC_R


def _p_specs():
    return [
        pl.BlockSpec((GC_R, F), lambda i: (i, 0)),
        pl.BlockSpec((GC_R, F), lambda i: (i + GC_GRID, 0)),
    ]


def _combine1_body(p0_ref, p1_ref, sl_ref, b_ref, o_ref):
    o_ref[...] = jnp.maximum(p0_ref[...] + p1_ref[...] + sl_ref[...] + b_ref[...], 0.0)


def _combine1(parts, s1l, b):
    return pl.pallas_call(
        _combine1_body,
        grid=(GC_GRID,),
        in_specs=_p_specs() + [
            pl.BlockSpec((GC_R, F), lambda i: (i, 0)),
            pl.BlockSpec((1, F), lambda i: (0, 0)),
        ],
        out_specs=pl.BlockSpec((GC_R, F), lambda i: (i, 0)),
        out_shape=jax.ShapeDtypeStruct((N_OUT, F), jnp.float32),
    )(parts, parts, s1l, b)


def _gconv_body(p0_ref, p1_ref, x_ref, w_ref, wl_ref, b_ref, o_ref, *, act):
    agg = p0_ref[...] + p1_ref[...]
    t = _mm3(agg, w_ref[...]) + _mm3(x_ref[...], wl_ref[...]) + b_ref[...]
    if act:
        t = jnp.maximum(t, 0.0)
    o_ref[...] = t


def _gconv_res_body(p0_ref, p1_ref, x_ref, hp_ref, w_ref, wl_ref, b_ref, o_ref):
    agg = p0_ref[...] + p1_ref[...]
    t = _mm3(agg, w_ref[...]) + _mm3(x_ref[...], wl_ref[...]) + b_ref[...]
    t = jnp.maximum(t, 0.0)
    o_ref[...] = (hp_ref[...] + t) * 0.5


def _gconv(parts, x, W, Wl, b, act=True, hprev=None):
    fo = W.shape[1]
    wspec = [
        pl.BlockSpec((F, fo), lambda i: (0, 0)),
        pl.BlockSpec((F, fo), lambda i: (0, 0)),
        pl.BlockSpec((1, fo), lambda i: (0, 0)),
    ]
    ospec = pl.BlockSpec((GC_R, fo), lambda i: (i, 0))
    oshape = jax.ShapeDtypeStruct((N_OUT, fo), jnp.float32)
    if hprev is None:
        return pl.pallas_call(
            functools.partial(_gconv_body, act=act),
            grid=(GC_GRID,),
            in_specs=_p_specs() + [pl.BlockSpec((GC_R, F), lambda i: (i, 0))] + wspec,
            out_specs=ospec,
            out_shape=oshape,
        )(parts, parts, x, W, Wl, b)
    return pl.pallas_call(
        _gconv_res_body,
        grid=(GC_GRID,),
        in_specs=_p_specs() + [
            pl.BlockSpec((GC_R, F), lambda i: (i, 0)),
            pl.BlockSpec((GC_R, F), lambda i: (i, 0)),
        ] + wspec,
        out_specs=ospec,
        out_shape=oshape,
    )(parts, parts, x, hprev, W, Wl, b)


def kernel(x, x2, x_hidden, edge_index, unpool_idx, Wq, Wk, Wv, c1_W, c1_Wl, c1_b,
           res_W, res_Wl, res_b, c2_W, c2_Wl, c2_b, out_W, out_Wl, out_b):
    src = edge_index[0]
    dst = edge_index[1]
    zeros640 = jnp.zeros((STRIPE_LAST, F), jnp.float32)

    mv = jnp.zeros((N_PAD, 8), jnp.float32).at[:N_IN, 1:4].set(x2[0])
    att = _attention(mv)(mv, mv, Wq, Wk, Wv)[:N_IN]

    cat = jnp.concatenate([x[0], x_hidden[0], att], axis=1)
    catp = jnp.pad(cat, ((0, N_PAD - N_IN), (0, 0)))
    y, yl = _pre(catp, c1_W, c1_Wl)

    u0 = jnp.pad(unpool_idx[:, 0], (0, N_PAD - N_IN))
    u1 = jnp.pad(unpool_idx[:, 1], (0, N_PAD - N_IN))
    ny, nyl = _unpool_sc(y, yl, u0, u1)
    s1 = jnp.concatenate([y[:N_IN], ny[:N_IN]], axis=0)
    s1l = jnp.concatenate([yl[:N_IN], nyl[:N_IN]], axis=0)

    def spmm(t):
        return _spmm_sc(t, src, dst, zeros640)

    h = _combine1(spmm(s1), s1l, c1_b[None, :])
    for i in range(6):
        t = _gconv(spmm(h), h, res_W[i, 0], res_Wl[i, 0], res_b[i, 0][None, :])
        h = _gconv(spmm(t), t, res_W[i, 1], res_Wl[i, 1], res_b[i, 1][None, :],
                   hprev=h)
    x4 = _gconv(spmm(h), h, c2_W, c2_Wl, c2_b[None, :])

    out_Wp = jnp.pad(out_W, ((0, 0), (0, F - out_W.shape[1])))
    out_Wlp = jnp.pad(out_Wl, ((0, 0), (0, F - out_Wl.shape[1])))
    out_bp = jnp.pad(out_b, (0, F - out_b.shape[0]))
    o = _gconv(spmm(x4), x4, out_Wp, out_Wlp, out_bp[None, :], act=False)
    return o[None, :, :3]
